# baseline (speedup 1.0000x reference)
import functools

import jax
import jax.numpy as jnp
from jax import lax
from jax.experimental import pallas as pl
from jax.experimental.pallas import tpu as pltpu
from jax.experimental.pallas import tpu_sc as plsc

F32 = jnp.float32
_PREC = jax.lax.Precision.DEFAULT

N = 10000
E = 320000
DF = 128
DE = 16
H = 128
L = 4
B = 16384

NC = 2
NS = 16
NW = NC * NS
EPW = E // NW
EC = 40
NCHUNK = EPW // EC
ZROWS = EC
NZCH = N // ZROWS
ZK = -(-NZCH // NS)
WROWS = 200
NWCH = N // WROWS
WK = -(-NWCH // NS)
BPW = B // NW
BC = 128
BCHUNK = BPW // BC

_VMESH = plsc.VectorSubcoreMesh(core_axis_name="c", subcore_axis_name="s")


def _matmul_bias_body(x_ref, w_ref, b_ref, o_ref):
    o_ref[...] = (
        jnp.dot(x_ref[...], w_ref[...], preferred_element_type=F32, precision=_PREC) + b_ref[...]
    )


def _node_encode(x, w, b):
    rb = 2000
    return pl.pallas_call(
        _matmul_bias_body,
        grid=(N // rb,),
        in_specs=[
            pl.BlockSpec((rb, DF), lambda i: (i, 0)),
            pl.BlockSpec((DF, H), lambda i: (0, 0)),
            pl.BlockSpec((1, H), lambda i: (0, 0)),
        ],
        out_specs=pl.BlockSpec((rb, H), lambda i: (i, 0)),
        out_shape=jax.ShapeDtypeStruct((N, H), F32),
    )(x, w, b.reshape(1, H))


def _ea_encode(edge_attr, ee_W, ee_b):
    eb = 8000
    return pl.pallas_call(
        _matmul_bias_body,
        grid=(E // eb,),
        in_specs=[
            pl.BlockSpec((eb, DE), lambda i: (i, 0)),
            pl.BlockSpec((DE, H), lambda i: (0, 0)),
            pl.BlockSpec((1, H), lambda i: (0, 0)),
        ],
        out_specs=pl.BlockSpec((eb, H), lambda i: (i, 0)),
        out_shape=jax.ShapeDtypeStruct((E, H), F32),
    )(edge_attr, ee_W, ee_b.reshape(1, H))


def _matmul_body(x_ref, w_ref, o_ref):
    o_ref[...] = jnp.dot(x_ref[...], w_ref[...], preferred_element_type=F32, precision=_PREC)


def _edge_term(ea, lin_W_l):
    eb = 4000
    return pl.pallas_call(
        _matmul_body,
        grid=(E // eb,),
        in_specs=[
            pl.BlockSpec((eb, H), lambda i: (i, 0)),
            pl.BlockSpec((H, H), lambda i: (0, 0)),
        ],
        out_specs=pl.BlockSpec((eb, H), lambda i: (i, 0)),
        out_shape=jax.ShapeDtypeStruct((E, H), F32),
    )(ea, lin_W_l)


def _node_update_body(x_ref, a_ref, w1_ref, b1_ref, w2_ref, b2_ref,
                      g_ref, be_ref, bg_ref, bb_ref, o_ref):
    h = x_ref[...] + (a_ref[0] + a_ref[1])
    t = jnp.maximum(jnp.dot(h, w1_ref[...], preferred_element_type=F32, precision=_PREC) + b1_ref[...], 0.0)
    h2 = jnp.dot(t, w2_ref[...], preferred_element_type=F32, precision=_PREC) + b2_ref[...]
    mu = jnp.mean(h2, axis=-1, keepdims=True)
    var = jnp.mean((h2 - mu) ** 2, axis=-1, keepdims=True)
    h2 = (h2 - mu) / jnp.sqrt(var + 1e-5) * g_ref[...] + be_ref[...]
    h2 = h2 / jnp.sqrt(1.0 + 1e-5) * bg_ref[...] + bb_ref[...]
    o_ref[...] = jnp.maximum(h2, 0.0)


def _node_update(x, aggr2, w1, b1, w2, b2, ln_g, ln_b, bn_g, bn_b):
    rb = 2000
    row = lambda a: a.reshape(1, H)
    return pl.pallas_call(
        _node_update_body,
        grid=(N // rb,),
        in_specs=[
            pl.BlockSpec((rb, H), lambda i: (i, 0)),
            pl.BlockSpec((2, rb, H), lambda i: (0, i, 0)),
            pl.BlockSpec((H, H), lambda i: (0, 0)),
            pl.BlockSpec((1, H), lambda i: (0, 0)),
            pl.BlockSpec((H, H), lambda i: (0, 0)),
            pl.BlockSpec((1, H), lambda i: (0, 0)),
            pl.BlockSpec((1, H), lambda i: (0, 0)),
            pl.BlockSpec((1, H), lambda i: (0, 0)),
            pl.BlockSpec((1, H), lambda i: (0, 0)),
            pl.BlockSpec((1, H), lambda i: (0, 0)),
        ],
        out_specs=pl.BlockSpec((rb, H), lambda i: (i, 0)),
        out_shape=jax.ShapeDtypeStruct((N, H), F32),
    )(x, aggr2, w1, row(b1), w2, row(b2), row(ln_g), row(ln_b), row(bn_g), row(bn_b))


def _head_body(e_ref, w1_ref, b1_ref, w2_ref, b2_ref, w3_ref, b3_ref, o_ref):
    z = jnp.maximum(jnp.dot(e_ref[...], w1_ref[...], preferred_element_type=F32, precision=_PREC) + b1_ref[...], 0.0)
    z = jnp.maximum(jnp.dot(z, w2_ref[...], preferred_element_type=F32, precision=_PREC) + b2_ref[...], 0.0)
    o_ref[...] = jnp.dot(z, w3_ref[...], preferred_element_type=F32, precision=_PREC) + b3_ref[...]


def _head(e, h1_W, h1_b, h2_W, h2_b, h3_W, h3_b):
    rb = 2048
    return pl.pallas_call(
        _head_body,
        grid=(B // rb,),
        in_specs=[
            pl.BlockSpec((rb, 2 * H), lambda i: (i, 0)),
            pl.BlockSpec((2 * H, H), lambda i: (0, 0)),
            pl.BlockSpec((1, H), lambda i: (0, 0)),
            pl.BlockSpec((H, H // 2), lambda i: (0, 0)),
            pl.BlockSpec((1, H // 2), lambda i: (0, 0)),
            pl.BlockSpec((H // 2, 1), lambda i: (0, 0)),
            pl.BlockSpec((1, 1), lambda i: (0, 0)),
        ],
        out_specs=pl.BlockSpec((rb, 1), lambda i: (i, 0)),
        out_shape=jax.ShapeDtypeStruct((B, 1), F32),
    )(e, h1_W, h1_b.reshape(1, H), h2_W, h2_b.reshape(1, H // 2),
      h3_W, h3_b.reshape(1, 1))


@functools.partial(
    pl.kernel,
    out_type=jax.ShapeDtypeStruct((NC, N, H), F32),
    mesh=_VMESH,
    scratch_types=[
        pltpu.VMEM((EPW,), jnp.int32),
        pltpu.VMEM((H,), F32),
        pltpu.VMEM((EC, H), F32),
        pltpu.VMEM((EC, H), F32),
        pltpu.VMEM((EC, H), F32),
        pltpu.VMEM((EC, H), F32),
        pltpu.VMEM((EC, H), F32),
        pltpu.VMEM((EC, H), F32),
        pltpu.VMEM((EC,), jnp.int32),
        pltpu.VMEM((EC,), jnp.int32),
        pltpu.VMEM_SHARED((N, H), F32),
        pltpu.SemaphoreType.DMA,
        pltpu.SemaphoreType.DMA,
        pltpu.SemaphoreType.DMA,
        pltpu.SemaphoreType.DMA,
        pltpu.SemaphoreType.DMA,
        pltpu.SemaphoreType.DMA,
        pltpu.SemaphoreType.DMA,
        pltpu.SemaphoreType.DMA,
    ],
)
def _sc_message(x_hbm, t_hbm, bias_hbm, src_hbm, dst_hbm, out_hbm,
                srcall, biasv, xg0, xg1, tv0, tv1, mb0, mb1, dstv0, dstv1,
                accum,
                gsem0, gsem1, tsem0, tsem1, ssem0, ssem1, dsem0, dsem1):
    c = lax.axis_index("c")
    s = lax.axis_index("s")
    wid = s * NC + c
    ebase = wid * EPW

    bufs = ((xg0, tv0, mb0, dstv0, gsem0, tsem0, ssem0, dsem0),
            (xg1, tv1, mb1, dstv1, gsem1, tsem1, ssem1, dsem1))

    def issue_loads(ci, b):
        xg, tv, _, _, gsem, tsem, _, _ = bufs[b]
        off = ci * EC
        pltpu.async_copy(x_hbm.at[srcall.at[pl.ds(off, EC)]], xg, gsem)
        pltpu.async_copy(t_hbm.at[pl.ds(ebase + off, EC), :], tv, tsem)

    def wait_loads(b):
        xg, tv, _, _, gsem, tsem, _, _ = bufs[b]
        pltpu.make_async_copy(x_hbm.at[srcall.at[pl.ds(0, EC)]], xg, gsem).wait()
        pltpu.make_async_copy(t_hbm.at[pl.ds(0, EC), :], tv, tsem).wait()

    def issue_dst(ci, b):
        _, _, _, dstv, _, _, _, dsem = bufs[b]
        pltpu.async_copy(dst_hbm.at[pl.ds(ebase + ci * EC, EC)], dstv, dsem)

    def wait_scatter(b):
        _, _, mb, dstv, _, _, ssem, _ = bufs[b]
        pltpu.make_async_copy(mb, accum.at[dstv], ssem).wait()

    def compute(b):
        xg, tv, mb, _, _, _, _, _ = bufs[b]

        @pl.loop(0, EC)
        def _(i):
            for j in range(H // 16):
                sl = pl.ds(j * 16, 16)
                mb[i, sl] = jnp.maximum((xg[i, sl] + tv[i, sl]) + biasv[sl], 0.0)

    def scatter(b):
        _, _, mb, dstv, _, _, ssem, dsem = bufs[b]
        pltpu.make_async_copy(dst_hbm.at[pl.ds(0, EC)], dstv, dsem).wait()
        pltpu.async_copy(mb, accum.at[dstv], ssem, add=True)

    pltpu.sync_copy(src_hbm.at[pl.ds(ebase, EPW)], srcall)
    pltpu.sync_copy(bias_hbm, biasv)

    @pl.loop(0, ZROWS)
    def _(i):
        for j in range(H // 16):
            xg0[i, pl.ds(j * 16, 16)] = jnp.zeros((16,), F32)

    @pl.loop(0, ZK)
    def _(k):
        cid = s + k * NS

        @pl.when(cid < NZCH)
        def _():
            pltpu.sync_copy(xg0, accum.at[pl.ds(cid * ZROWS, ZROWS), :])

    plsc.subcore_barrier()

    issue_loads(0, 0)
    issue_loads(1, 1)

    @pl.loop(0, NCHUNK // 2)
    def _(k):
        for b in range(2):
            ci = 2 * k + b
            wait_loads(b)

            @pl.when(k > 0)
            def _():
                wait_scatter(b)

            issue_dst(ci, b)
            compute(b)
            nxt = ci + 2

            @pl.when(nxt < NCHUNK)
            def _():
                issue_loads(nxt, b)

            scatter(b)

    wait_scatter(0)
    wait_scatter(1)

    plsc.subcore_barrier()

    @pl.loop(0, WK)
    def _(k):
        cid = s + k * NS

        @pl.when(cid < NWCH)
        def _():
            r0 = cid * WROWS
            pltpu.sync_copy(accum.at[pl.ds(r0, WROWS), :],
                            out_hbm.at[c, pl.ds(r0, WROWS), :])


@functools.partial(
    pl.kernel,
    out_type=jax.ShapeDtypeStruct((B, 2 * H), F32),
    mesh=_VMESH,
    scratch_types=[
        pltpu.VMEM((BC,), jnp.int32),
        pltpu.VMEM((BC,), jnp.int32),
        pltpu.VMEM((BC, H), F32),
        pltpu.VMEM((BC, H), F32),
        pltpu.SemaphoreType.DMA,
        pltpu.SemaphoreType.DMA,
        pltpu.SemaphoreType.DMA,
    ],
)
def _sc_pair_gather(x_hbm, sidx_hbm, tidx_hbm, out_hbm,
                    sidxv, tidxv, ug, vg, gsem, osem0, osem1):
    c = lax.axis_index("c")
    s = lax.axis_index("s")
    wid = s * NC + c

    @pl.loop(0, BCHUNK)
    def _(ci):
        base = wid * BPW + ci * BC
        pltpu.sync_copy(sidx_hbm.at[pl.ds(base, BC)], sidxv)
        pltpu.sync_copy(tidx_hbm.at[pl.ds(base, BC)], tidxv)
        pltpu.async_copy(x_hbm.at[sidxv], ug, gsem).wait()
        pltpu.async_copy(x_hbm.at[tidxv], vg, gsem).wait()
        pltpu.async_copy(ug, out_hbm.at[pl.ds(base, BC), pl.ds(0, H)], osem0)
        pltpu.async_copy(vg, out_hbm.at[pl.ds(base, BC), pl.ds(H, H)], osem1)
        pltpu.make_async_copy(ug, out_hbm.at[pl.ds(base, BC), pl.ds(0, H)], osem0).wait()
        pltpu.make_async_copy(vg, out_hbm.at[pl.ds(base, BC), pl.ds(H, H)], osem1).wait()


def kernel(node_embeddings, edge_index, edge_attr, target_edges_tensor,
           ne_W, ne_b, ee_W, ee_b, lin_W, lin_b, mlp_W1, mlp_b1, mlp_W2,
           mlp_b2, ln_g, ln_b, bn_g, bn_b, h1_W, h1_b, h2_W, h2_b, h3_W, h3_b):
    src = edge_index[0]
    dst = edge_index[1]
    sidx = target_edges_tensor[:, 0]
    tidx = target_edges_tensor[:, 1]

    x = _node_encode(node_embeddings, ne_W, ne_b)
    ea = _ea_encode(edge_attr, ee_W, ee_b)

    for l in range(L):
        t = _edge_term(ea, lin_W[l])
        aggr2 = _sc_message(x, t, lin_b[l], src, dst)
        x = _node_update(x, aggr2, mlp_W1[l], mlp_b1[l], mlp_W2[l], mlp_b2[l],
                         ln_g[l], ln_b[l], bn_g[l], bn_b[l])

    e = _sc_pair_gather(x, sidx, tidx)
    logits = _head(e, h1_W, h1_b, h2_W, h2_b, h3_W, h3_b)
    return logits[:, 0]


# speedup vs baseline: 1.0265x; 1.0265x over previous
import functools

import jax
import jax.numpy as jnp
from jax import lax
from jax.experimental import pallas as pl
from jax.experimental.pallas import tpu as pltpu
from jax.experimental.pallas import tpu_sc as plsc

F32 = jnp.float32
_PREC = jax.lax.Precision.DEFAULT

N = 10000
E = 320000
DF = 128
DE = 16
H = 128
L = 4
B = 16384

NC = 2
NS = 16
NW = NC * NS
EPW = E // NW
EC = 40
NCHUNK = EPW // EC
ZROWS = EC
NZCH = N // ZROWS
ZK = -(-NZCH // NS)
WROWS = 200
NWCH = N // WROWS
WK = -(-NWCH // NS)
BPW = B // NW
BC = 128
BCHUNK = BPW // BC

_VMESH = plsc.VectorSubcoreMesh(core_axis_name="c", subcore_axis_name="s")


def _matmul_bias_body(x_ref, w_ref, b_ref, o_ref):
    o_ref[...] = (
        jnp.dot(x_ref[...], w_ref[...], preferred_element_type=F32, precision=_PREC) + b_ref[...]
    )


def _node_encode(x, w, b):
    rb = 2000
    return pl.pallas_call(
        _matmul_bias_body,
        grid=(N // rb,),
        in_specs=[
            pl.BlockSpec((rb, DF), lambda i: (i, 0)),
            pl.BlockSpec((DF, H), lambda i: (0, 0)),
            pl.BlockSpec((1, H), lambda i: (0, 0)),
        ],
        out_specs=pl.BlockSpec((rb, H), lambda i: (i, 0)),
        out_shape=jax.ShapeDtypeStruct((N, H), F32),
    )(x, w, b.reshape(1, H))


def _ea_encode_body(x_ref, w_ref, b_ref, o_ref):
    o_ref[...] = (
        jnp.dot(x_ref[...], w_ref[...], preferred_element_type=F32, precision=_PREC)
        + b_ref[...]
    ).astype(jnp.bfloat16)


def _ea_encode(edge_attr, ee_W, ee_b):
    eb = 8000
    return pl.pallas_call(
        _ea_encode_body,
        grid=(E // eb,),
        in_specs=[
            pl.BlockSpec((eb, DE), lambda i: (i, 0)),
            pl.BlockSpec((DE, H), lambda i: (0, 0)),
            pl.BlockSpec((1, H), lambda i: (0, 0)),
        ],
        out_specs=pl.BlockSpec((eb, H), lambda i: (i, 0)),
        out_shape=jax.ShapeDtypeStruct((E, H), jnp.bfloat16),
    )(edge_attr, ee_W, ee_b.reshape(1, H))


def _matmul_body(x_ref, w_ref, o_ref):
    o_ref[...] = jnp.dot(x_ref[...], w_ref[...], preferred_element_type=F32, precision=_PREC)


def _edge_term(ea_bf16, lin_W_l_bf16):
    eb = 4000
    return pl.pallas_call(
        _matmul_body,
        grid=(E // eb,),
        in_specs=[
            pl.BlockSpec((eb, H), lambda i: (i, 0)),
            pl.BlockSpec((H, H), lambda i: (0, 0)),
        ],
        out_specs=pl.BlockSpec((eb, H), lambda i: (i, 0)),
        out_shape=jax.ShapeDtypeStruct((E, H), F32),
    )(ea_bf16, lin_W_l_bf16)


def _node_update_body(x_ref, a_ref, w1_ref, b1_ref, w2_ref, b2_ref,
                      g_ref, be_ref, bg_ref, bb_ref, o_ref):
    h = x_ref[...] + (a_ref[0] + a_ref[1])
    t = jnp.maximum(jnp.dot(h, w1_ref[...], preferred_element_type=F32, precision=_PREC) + b1_ref[...], 0.0)
    h2 = jnp.dot(t, w2_ref[...], preferred_element_type=F32, precision=_PREC) + b2_ref[...]
    mu = jnp.mean(h2, axis=-1, keepdims=True)
    var = jnp.mean((h2 - mu) ** 2, axis=-1, keepdims=True)
    h2 = (h2 - mu) / jnp.sqrt(var + 1e-5) * g_ref[...] + be_ref[...]
    h2 = h2 / jnp.sqrt(1.0 + 1e-5) * bg_ref[...] + bb_ref[...]
    o_ref[...] = jnp.maximum(h2, 0.0)


def _node_update(x, aggr2, w1, b1, w2, b2, ln_g, ln_b, bn_g, bn_b):
    rb = 2000
    row = lambda a: a.reshape(1, H)
    return pl.pallas_call(
        _node_update_body,
        grid=(N // rb,),
        in_specs=[
            pl.BlockSpec((rb, H), lambda i: (i, 0)),
            pl.BlockSpec((2, rb, H), lambda i: (0, i, 0)),
            pl.BlockSpec((H, H), lambda i: (0, 0)),
            pl.BlockSpec((1, H), lambda i: (0, 0)),
            pl.BlockSpec((H, H), lambda i: (0, 0)),
            pl.BlockSpec((1, H), lambda i: (0, 0)),
            pl.BlockSpec((1, H), lambda i: (0, 0)),
            pl.BlockSpec((1, H), lambda i: (0, 0)),
            pl.BlockSpec((1, H), lambda i: (0, 0)),
            pl.BlockSpec((1, H), lambda i: (0, 0)),
        ],
        out_specs=pl.BlockSpec((rb, H), lambda i: (i, 0)),
        out_shape=jax.ShapeDtypeStruct((N, H), F32),
    )(x, aggr2, w1, row(b1), w2, row(b2), row(ln_g), row(ln_b), row(bn_g), row(bn_b))


def _head_body(e_ref, w1_ref, b1_ref, w2_ref, b2_ref, w3_ref, b3_ref, o_ref):
    z = jnp.maximum(jnp.dot(e_ref[...], w1_ref[...], preferred_element_type=F32, precision=_PREC) + b1_ref[...], 0.0)
    z = jnp.maximum(jnp.dot(z, w2_ref[...], preferred_element_type=F32, precision=_PREC) + b2_ref[...], 0.0)
    o_ref[...] = jnp.dot(z, w3_ref[...], preferred_element_type=F32, precision=_PREC) + b3_ref[...]


def _head(e, h1_W, h1_b, h2_W, h2_b, h3_W, h3_b):
    rb = 2048
    return pl.pallas_call(
        _head_body,
        grid=(B // rb,),
        in_specs=[
            pl.BlockSpec((rb, 2 * H), lambda i: (i, 0)),
            pl.BlockSpec((2 * H, H), lambda i: (0, 0)),
            pl.BlockSpec((1, H), lambda i: (0, 0)),
            pl.BlockSpec((H, H // 2), lambda i: (0, 0)),
            pl.BlockSpec((1, H // 2), lambda i: (0, 0)),
            pl.BlockSpec((H // 2, 1), lambda i: (0, 0)),
            pl.BlockSpec((1, 1), lambda i: (0, 0)),
        ],
        out_specs=pl.BlockSpec((rb, 1), lambda i: (i, 0)),
        out_shape=jax.ShapeDtypeStruct((B, 1), F32),
    )(e, h1_W, h1_b.reshape(1, H), h2_W, h2_b.reshape(1, H // 2),
      h3_W, h3_b.reshape(1, 1))


@functools.partial(
    pl.kernel,
    out_type=jax.ShapeDtypeStruct((NC, N, H), F32),
    mesh=_VMESH,
    scratch_types=[
        pltpu.VMEM((EPW,), jnp.int32),
        pltpu.VMEM((H,), F32),
        pltpu.VMEM((EC, H), F32),
        pltpu.VMEM((EC, H), F32),
        pltpu.VMEM((EC, H), F32),
        pltpu.VMEM((EC, H), F32),
        pltpu.VMEM((EC, H), F32),
        pltpu.VMEM((EC, H), F32),
        pltpu.VMEM((EC,), jnp.int32),
        pltpu.VMEM((EC,), jnp.int32),
        pltpu.VMEM_SHARED((N, H), F32),
        pltpu.SemaphoreType.DMA,
        pltpu.SemaphoreType.DMA,
        pltpu.SemaphoreType.DMA,
        pltpu.SemaphoreType.DMA,
        pltpu.SemaphoreType.DMA,
        pltpu.SemaphoreType.DMA,
        pltpu.SemaphoreType.DMA,
        pltpu.SemaphoreType.DMA,
    ],
)
def _sc_message(x_hbm, t_hbm, bias_hbm, src_hbm, dst_hbm, out_hbm,
                srcall, biasv, xg0, xg1, tv0, tv1, mb0, mb1, dstv0, dstv1,
                accum,
                gsem0, gsem1, tsem0, tsem1, ssem0, ssem1, dsem0, dsem1):
    c = lax.axis_index("c")
    s = lax.axis_index("s")
    wid = s * NC + c
    ebase = wid * EPW

    bufs = ((xg0, tv0, mb0, dstv0, gsem0, tsem0, ssem0, dsem0),
            (xg1, tv1, mb1, dstv1, gsem1, tsem1, ssem1, dsem1))

    def issue_loads(ci, b):
        xg, tv, _, _, gsem, tsem, _, _ = bufs[b]
        off = ci * EC
        pltpu.async_copy(x_hbm.at[srcall.at[pl.ds(off, EC)]], xg, gsem)
        pltpu.async_copy(t_hbm.at[pl.ds(ebase + off, EC), :], tv, tsem)

    def wait_loads(b):
        xg, tv, _, _, gsem, tsem, _, _ = bufs[b]
        pltpu.make_async_copy(x_hbm.at[srcall.at[pl.ds(0, EC)]], xg, gsem).wait()
        pltpu.make_async_copy(t_hbm.at[pl.ds(0, EC), :], tv, tsem).wait()

    def issue_dst(ci, b):
        _, _, _, dstv, _, _, _, dsem = bufs[b]
        pltpu.async_copy(dst_hbm.at[pl.ds(ebase + ci * EC, EC)], dstv, dsem)

    def wait_scatter(b):
        _, _, mb, dstv, _, _, ssem, _ = bufs[b]
        pltpu.make_async_copy(mb, accum.at[dstv], ssem).wait()

    def compute(b):
        xg, tv, mb, _, _, _, _, _ = bufs[b]

        @pl.loop(0, EC)
        def _(i):
            for j in range(H // 16):
                sl = pl.ds(j * 16, 16)
                mb[i, sl] = jnp.maximum((xg[i, sl] + tv[i, sl]) + biasv[sl], 0.0)

    def scatter(b):
        _, _, mb, dstv, _, _, ssem, dsem = bufs[b]
        pltpu.make_async_copy(dst_hbm.at[pl.ds(0, EC)], dstv, dsem).wait()
        pltpu.async_copy(mb, accum.at[dstv], ssem, add=True)

    pltpu.sync_copy(src_hbm.at[pl.ds(ebase, EPW)], srcall)
    pltpu.sync_copy(bias_hbm, biasv)

    @pl.loop(0, ZROWS)
    def _(i):
        for j in range(H // 16):
            xg0[i, pl.ds(j * 16, 16)] = jnp.zeros((16,), F32)

    @pl.loop(0, ZK)
    def _(k):
        cid = s + k * NS

        @pl.when(cid < NZCH)
        def _():
            pltpu.sync_copy(xg0, accum.at[pl.ds(cid * ZROWS, ZROWS), :])

    plsc.subcore_barrier()

    issue_loads(0, 0)
    issue_loads(1, 1)

    @pl.loop(0, NCHUNK // 2)
    def _(k):
        for b in range(2):
            ci = 2 * k + b
            wait_loads(b)

            @pl.when(k > 0)
            def _():
                wait_scatter(b)

            issue_dst(ci, b)
            compute(b)
            nxt = ci + 2

            @pl.when(nxt < NCHUNK)
            def _():
                issue_loads(nxt, b)

            scatter(b)

    wait_scatter(0)
    wait_scatter(1)

    plsc.subcore_barrier()

    @pl.loop(0, WK)
    def _(k):
        cid = s + k * NS

        @pl.when(cid < NWCH)
        def _():
            r0 = cid * WROWS
            pltpu.sync_copy(accum.at[pl.ds(r0, WROWS), :],
                            out_hbm.at[c, pl.ds(r0, WROWS), :])


@functools.partial(
    pl.kernel,
    out_type=jax.ShapeDtypeStruct((B, 2 * H), F32),
    mesh=_VMESH,
    scratch_types=[
        pltpu.VMEM((BC,), jnp.int32),
        pltpu.VMEM((BC,), jnp.int32),
        pltpu.VMEM((BC, H), F32),
        pltpu.VMEM((BC, H), F32),
        pltpu.SemaphoreType.DMA,
        pltpu.SemaphoreType.DMA,
        pltpu.SemaphoreType.DMA,
    ],
)
def _sc_pair_gather(x_hbm, sidx_hbm, tidx_hbm, out_hbm,
                    sidxv, tidxv, ug, vg, gsem, osem0, osem1):
    c = lax.axis_index("c")
    s = lax.axis_index("s")
    wid = s * NC + c

    @pl.loop(0, BCHUNK)
    def _(ci):
        base = wid * BPW + ci * BC
        pltpu.sync_copy(sidx_hbm.at[pl.ds(base, BC)], sidxv)
        pltpu.sync_copy(tidx_hbm.at[pl.ds(base, BC)], tidxv)
        pltpu.async_copy(x_hbm.at[sidxv], ug, gsem).wait()
        pltpu.async_copy(x_hbm.at[tidxv], vg, gsem).wait()
        pltpu.async_copy(ug, out_hbm.at[pl.ds(base, BC), pl.ds(0, H)], osem0)
        pltpu.async_copy(vg, out_hbm.at[pl.ds(base, BC), pl.ds(H, H)], osem1)
        pltpu.make_async_copy(ug, out_hbm.at[pl.ds(base, BC), pl.ds(0, H)], osem0).wait()
        pltpu.make_async_copy(vg, out_hbm.at[pl.ds(base, BC), pl.ds(H, H)], osem1).wait()


def kernel(node_embeddings, edge_index, edge_attr, target_edges_tensor,
           ne_W, ne_b, ee_W, ee_b, lin_W, lin_b, mlp_W1, mlp_b1, mlp_W2,
           mlp_b2, ln_g, ln_b, bn_g, bn_b, h1_W, h1_b, h2_W, h2_b, h3_W, h3_b):
    src = edge_index[0]
    dst = edge_index[1]
    sidx = target_edges_tensor[:, 0]
    tidx = target_edges_tensor[:, 1]

    x = _node_encode(node_embeddings, ne_W, ne_b)
    ea = _ea_encode(edge_attr, ee_W, ee_b)

    lin_W_bf16 = lin_W.astype(jnp.bfloat16)
    for l in range(L):
        t = _edge_term(ea, lin_W_bf16[l])
        aggr2 = _sc_message(x, t, lin_b[l], src, dst)
        x = _node_update(x, aggr2, mlp_W1[l], mlp_b1[l], mlp_W2[l], mlp_b2[l],
                         ln_g[l], ln_b[l], bn_g[l], bn_b[l])

    e = _sc_pair_gather(x, sidx, tidx)
    logits = _head(e, h1_W, h1_b, h2_W, h2_b, h3_W, h3_b)
    return logits[:, 0]


# speedup vs baseline: 1.6893x; 1.6457x over previous
import functools

import jax
import jax.numpy as jnp
from jax import lax
from jax.experimental import pallas as pl
from jax.experimental.pallas import tpu as pltpu
from jax.experimental.pallas import tpu_sc as plsc

F32 = jnp.float32
_PREC = jax.lax.Precision.DEFAULT

N = 10000
E = 320000
DF = 128
DE = 16
H = 128
L = 4
B = 16384

NC = 2
NS = 16
NW = NC * NS
EPW = E // NW
EC = 40
NCHUNK = EPW // EC
ZROWS = EC
NZCH = N // ZROWS
ZK = -(-NZCH // NS)
WROWS = 200
NWCH = N // WROWS
WK = -(-NWCH // NS)
BPW = B // NW
BC = 128
BCHUNK = BPW // BC

_VMESH = plsc.VectorSubcoreMesh(core_axis_name="c", subcore_axis_name="s")


def _matmul_bias_body(x_ref, w_ref, b_ref, o_ref):
    o_ref[...] = (
        jnp.dot(x_ref[...], w_ref[...], preferred_element_type=F32, precision=_PREC) + b_ref[...]
    )


def _node_encode(x, w, b):
    rb = 2000
    return pl.pallas_call(
        _matmul_bias_body,
        grid=(N // rb,),
        in_specs=[
            pl.BlockSpec((rb, DF), lambda i: (i, 0)),
            pl.BlockSpec((DF, H), lambda i: (0, 0)),
            pl.BlockSpec((1, H), lambda i: (0, 0)),
        ],
        out_specs=pl.BlockSpec((rb, H), lambda i: (i, 0)),
        out_shape=jax.ShapeDtypeStruct((N, H), F32),
    )(x, w, b.reshape(1, H))


def _ea_encode_body(x_ref, w_ref, b_ref, o_ref):
    o_ref[...] = (
        jnp.dot(x_ref[...], w_ref[...], preferred_element_type=F32, precision=_PREC)
        + b_ref[...]
    ).astype(jnp.bfloat16)


def _ea_encode(edge_attr, ee_W, ee_b):
    eb = 8000
    return pl.pallas_call(
        _ea_encode_body,
        grid=(E // eb,),
        in_specs=[
            pl.BlockSpec((eb, DE), lambda i: (i, 0)),
            pl.BlockSpec((DE, H), lambda i: (0, 0)),
            pl.BlockSpec((1, H), lambda i: (0, 0)),
        ],
        out_specs=pl.BlockSpec((eb, H), lambda i: (i, 0)),
        out_shape=jax.ShapeDtypeStruct((E, H), jnp.bfloat16),
    )(edge_attr, ee_W, ee_b.reshape(1, H))


def _matmul_body(x_ref, w_ref, o_ref):
    o_ref[...] = jnp.dot(x_ref[...], w_ref[...], preferred_element_type=F32, precision=_PREC)


def _edge_term(ea_bf16, lin_W_l_bf16, lin_b_l):
    eb = 4000
    return pl.pallas_call(
        _matmul_bias_body,
        grid=(E // eb,),
        in_specs=[
            pl.BlockSpec((eb, H), lambda i: (i, 0)),
            pl.BlockSpec((H, H), lambda i: (0, 0)),
            pl.BlockSpec((1, H), lambda i: (0, 0)),
        ],
        out_specs=pl.BlockSpec((eb, H), lambda i: (i, 0)),
        out_shape=jax.ShapeDtypeStruct((E, H), F32),
    )(ea_bf16, lin_W_l_bf16, lin_b_l.reshape(1, H))


def _node_update_body(x_ref, a_ref, w1_ref, b1_ref, w2_ref, b2_ref,
                      g_ref, be_ref, bg_ref, bb_ref, o_ref):
    h = x_ref[...] + (a_ref[0] + a_ref[1])
    t = jnp.maximum(jnp.dot(h, w1_ref[...], preferred_element_type=F32, precision=_PREC) + b1_ref[...], 0.0)
    h2 = jnp.dot(t, w2_ref[...], preferred_element_type=F32, precision=_PREC) + b2_ref[...]
    mu = jnp.mean(h2, axis=-1, keepdims=True)
    var = jnp.mean((h2 - mu) ** 2, axis=-1, keepdims=True)
    h2 = (h2 - mu) / jnp.sqrt(var + 1e-5) * g_ref[...] + be_ref[...]
    h2 = h2 / jnp.sqrt(1.0 + 1e-5) * bg_ref[...] + bb_ref[...]
    o_ref[...] = jnp.maximum(h2, 0.0)


def _node_update(x, aggr2, w1, b1, w2, b2, ln_g, ln_b, bn_g, bn_b):
    rb = 2000
    row = lambda a: a.reshape(1, H)
    return pl.pallas_call(
        _node_update_body,
        grid=(N // rb,),
        in_specs=[
            pl.BlockSpec((rb, H), lambda i: (i, 0)),
            pl.BlockSpec((2, rb, H), lambda i: (0, i, 0)),
            pl.BlockSpec((H, H), lambda i: (0, 0)),
            pl.BlockSpec((1, H), lambda i: (0, 0)),
            pl.BlockSpec((H, H), lambda i: (0, 0)),
            pl.BlockSpec((1, H), lambda i: (0, 0)),
            pl.BlockSpec((1, H), lambda i: (0, 0)),
            pl.BlockSpec((1, H), lambda i: (0, 0)),
            pl.BlockSpec((1, H), lambda i: (0, 0)),
            pl.BlockSpec((1, H), lambda i: (0, 0)),
        ],
        out_specs=pl.BlockSpec((rb, H), lambda i: (i, 0)),
        out_shape=jax.ShapeDtypeStruct((N, H), F32),
    )(x, aggr2, w1, row(b1), w2, row(b2), row(ln_g), row(ln_b), row(bn_g), row(bn_b))


def _head_body(e_ref, w1_ref, b1_ref, w2_ref, b2_ref, w3_ref, b3_ref, o_ref):
    z = jnp.maximum(jnp.dot(e_ref[...], w1_ref[...], preferred_element_type=F32, precision=_PREC) + b1_ref[...], 0.0)
    z = jnp.maximum(jnp.dot(z, w2_ref[...], preferred_element_type=F32, precision=_PREC) + b2_ref[...], 0.0)
    o_ref[...] = jnp.dot(z, w3_ref[...], preferred_element_type=F32, precision=_PREC) + b3_ref[...]


def _head(e, h1_W, h1_b, h2_W, h2_b, h3_W, h3_b):
    rb = 2048
    return pl.pallas_call(
        _head_body,
        grid=(B // rb,),
        in_specs=[
            pl.BlockSpec((rb, 2 * H), lambda i: (i, 0)),
            pl.BlockSpec((2 * H, H), lambda i: (0, 0)),
            pl.BlockSpec((1, H), lambda i: (0, 0)),
            pl.BlockSpec((H, H // 2), lambda i: (0, 0)),
            pl.BlockSpec((1, H // 2), lambda i: (0, 0)),
            pl.BlockSpec((H // 2, 1), lambda i: (0, 0)),
            pl.BlockSpec((1, 1), lambda i: (0, 0)),
        ],
        out_specs=pl.BlockSpec((rb, 1), lambda i: (i, 0)),
        out_shape=jax.ShapeDtypeStruct((B, 1), F32),
    )(e, h1_W, h1_b.reshape(1, H), h2_W, h2_b.reshape(1, H // 2),
      h3_W, h3_b.reshape(1, 1))


@functools.partial(
    pl.kernel,
    out_type=jax.ShapeDtypeStruct((NC, N, H), F32),
    mesh=_VMESH,
    scratch_types=[
        pltpu.VMEM((EPW,), jnp.int32),
        pltpu.VMEM((EC, H), F32),
        pltpu.VMEM((EC, H), F32),
        pltpu.VMEM((EC, H), F32),
        pltpu.VMEM((EC, H), F32),
        pltpu.VMEM((EC, H), F32),
        pltpu.VMEM((EC, H), F32),
        pltpu.VMEM((EC,), jnp.int32),
        pltpu.VMEM((EC,), jnp.int32),
        pltpu.VMEM_SHARED((N, H), F32),
        pltpu.SemaphoreType.DMA,
        pltpu.SemaphoreType.DMA,
        pltpu.SemaphoreType.DMA,
        pltpu.SemaphoreType.DMA,
        pltpu.SemaphoreType.DMA,
        pltpu.SemaphoreType.DMA,
        pltpu.SemaphoreType.DMA,
        pltpu.SemaphoreType.DMA,
    ],
)
def _sc_message(x_hbm, t_hbm, src_hbm, dst_hbm, out_hbm,
                srcall, xg0, xg1, tv0, tv1, mb0, mb1, dstv0, dstv1,
                accum,
                gsem0, gsem1, tsem0, tsem1, ssem0, ssem1, dsem0, dsem1):
    c = lax.axis_index("c")
    s = lax.axis_index("s")
    wid = s * NC + c
    ebase = wid * EPW

    bufs = ((xg0, tv0, mb0, dstv0, gsem0, tsem0, ssem0, dsem0),
            (xg1, tv1, mb1, dstv1, gsem1, tsem1, ssem1, dsem1))

    def issue_loads(ci, b):
        xg, tv, _, _, gsem, tsem, _, _ = bufs[b]
        off = ci * EC
        pltpu.async_copy(x_hbm.at[srcall.at[pl.ds(off, EC)]], xg, gsem)
        pltpu.async_copy(t_hbm.at[pl.ds(ebase + off, EC), :], tv, tsem)

    def wait_loads(b):
        xg, tv, _, _, gsem, tsem, _, _ = bufs[b]
        pltpu.make_async_copy(x_hbm.at[srcall.at[pl.ds(0, EC)]], xg, gsem).wait()
        pltpu.make_async_copy(t_hbm.at[pl.ds(0, EC), :], tv, tsem).wait()

    def issue_dst(ci, b):
        _, _, _, dstv, _, _, _, dsem = bufs[b]
        pltpu.async_copy(dst_hbm.at[pl.ds(ebase + ci * EC, EC)], dstv, dsem)

    def wait_scatter(b):
        _, _, mb, dstv, _, _, ssem, _ = bufs[b]
        pltpu.make_async_copy(mb, accum.at[dstv], ssem).wait()

    def compute(b):
        xg, tv, mb, _, _, _, _, _ = bufs[b]

        @pl.loop(0, EC)
        def _(i):
            for j in range(H // 16):
                sl = pl.ds(j * 16, 16)
                mb[i, sl] = jnp.maximum(xg[i, sl] + tv[i, sl], 0.0)

    def scatter(b):
        _, _, mb, dstv, _, _, ssem, dsem = bufs[b]
        pltpu.make_async_copy(dst_hbm.at[pl.ds(0, EC)], dstv, dsem).wait()
        pltpu.async_copy(mb, accum.at[dstv], ssem, add=True)

    pltpu.sync_copy(src_hbm.at[pl.ds(ebase, EPW)], srcall)

    @pl.loop(0, ZROWS)
    def _(i):
        for j in range(H // 16):
            xg0[i, pl.ds(j * 16, 16)] = jnp.zeros((16,), F32)

    @pl.loop(0, ZK)
    def _(k):
        cid = s + k * NS

        @pl.when(cid < NZCH)
        def _():
            pltpu.sync_copy(xg0, accum.at[pl.ds(cid * ZROWS, ZROWS), :])

    plsc.subcore_barrier()

    issue_loads(0, 0)
    issue_loads(1, 1)

    @pl.loop(0, NCHUNK // 2)
    def _(k):
        for b in range(2):
            ci = 2 * k + b
            wait_loads(b)

            @pl.when(k > 0)
            def _():
                wait_scatter(b)

            issue_dst(ci, b)
            compute(b)
            nxt = ci + 2

            @pl.when(nxt < NCHUNK)
            def _():
                issue_loads(nxt, b)

            scatter(b)

    wait_scatter(0)
    wait_scatter(1)

    plsc.subcore_barrier()

    @pl.loop(0, WK)
    def _(k):
        cid = s + k * NS

        @pl.when(cid < NWCH)
        def _():
            r0 = cid * WROWS
            pltpu.sync_copy(accum.at[pl.ds(r0, WROWS), :],
                            out_hbm.at[c, pl.ds(r0, WROWS), :])


@functools.partial(
    pl.kernel,
    out_type=jax.ShapeDtypeStruct((B, 2 * H), F32),
    mesh=_VMESH,
    scratch_types=[
        pltpu.VMEM((BC,), jnp.int32),
        pltpu.VMEM((BC,), jnp.int32),
        pltpu.VMEM((BC, H), F32),
        pltpu.VMEM((BC, H), F32),
        pltpu.SemaphoreType.DMA,
        pltpu.SemaphoreType.DMA,
        pltpu.SemaphoreType.DMA,
    ],
)
def _sc_pair_gather(x_hbm, sidx_hbm, tidx_hbm, out_hbm,
                    sidxv, tidxv, ug, vg, gsem, osem0, osem1):
    c = lax.axis_index("c")
    s = lax.axis_index("s")
    wid = s * NC + c

    @pl.loop(0, BCHUNK)
    def _(ci):
        base = wid * BPW + ci * BC
        pltpu.sync_copy(sidx_hbm.at[pl.ds(base, BC)], sidxv)
        pltpu.sync_copy(tidx_hbm.at[pl.ds(base, BC)], tidxv)
        pltpu.async_copy(x_hbm.at[sidxv], ug, gsem).wait()
        pltpu.async_copy(x_hbm.at[tidxv], vg, gsem).wait()
        pltpu.async_copy(ug, out_hbm.at[pl.ds(base, BC), pl.ds(0, H)], osem0)
        pltpu.async_copy(vg, out_hbm.at[pl.ds(base, BC), pl.ds(H, H)], osem1)
        pltpu.make_async_copy(ug, out_hbm.at[pl.ds(base, BC), pl.ds(0, H)], osem0).wait()
        pltpu.make_async_copy(vg, out_hbm.at[pl.ds(base, BC), pl.ds(H, H)], osem1).wait()


def kernel(node_embeddings, edge_index, edge_attr, target_edges_tensor,
           ne_W, ne_b, ee_W, ee_b, lin_W, lin_b, mlp_W1, mlp_b1, mlp_W2,
           mlp_b2, ln_g, ln_b, bn_g, bn_b, h1_W, h1_b, h2_W, h2_b, h3_W, h3_b):
    src = edge_index[0]
    dst = edge_index[1]
    sidx = target_edges_tensor[:, 0]
    tidx = target_edges_tensor[:, 1]

    x = _node_encode(node_embeddings, ne_W, ne_b)
    ea = _ea_encode(edge_attr, ee_W, ee_b)

    lin_W_bf16 = lin_W.astype(jnp.bfloat16)
    for l in range(L):
        t = _edge_term(ea, lin_W_bf16[l], lin_b[l])
        aggr2 = _sc_message(x, t, src, dst)
        x = _node_update(x, aggr2, mlp_W1[l], mlp_b1[l], mlp_W2[l], mlp_b2[l],
                         ln_g[l], ln_b[l], bn_g[l], bn_b[l])

    e = _sc_pair_gather(x, sidx, tidx)
    logits = _head(e, h1_W, h1_b, h2_W, h2_b, h3_W, h3_b)
    return logits[:, 0]
